# Initial kernel scaffold; baseline (speedup 1.0000x reference)
import jax
import jax.numpy as jnp
from jax.experimental import pallas as pl


def kernel(shengmu_indices, yunmu_indices, shengmu_table, yunmu_table):
    raise NotImplementedError("write your pallas kernel here")


# speedup vs baseline: 6.6099x; 6.6099x over previous
import functools

import jax
import jax.numpy as jnp
from jax import lax
from jax.experimental import pallas as pl
from jax.experimental.pallas import tpu as pltpu
from jax.experimental.pallas import tpu_sc as plsc

SH_V, YU_V = 24, 40
SH_D, YU_D = 32, 32
OUT_D = SH_D + YU_D
TAB_ROWS = SH_V * YU_V
NC, NS, L = 2, 16, 16
NW = NC * NS
BLK = 128


def _build_table_body(sh_ref, yu_ref, out_ref):
    sh = sh_ref[...]
    yu = yu_ref[...]
    shb = jnp.broadcast_to(sh[:, None, :], (SH_V, YU_V, SH_D)).reshape(
        TAB_ROWS, SH_D)
    yub = jnp.broadcast_to(yu[None, :, :], (SH_V, YU_V, YU_D)).reshape(
        TAB_ROWS, YU_D)
    out_ref[...] = jnp.concatenate([shb, yub], axis=-1)


def _build_table(sh_table, yu_table):
    return pl.pallas_call(
        _build_table_body,
        out_shape=jax.ShapeDtypeStruct((TAB_ROWS, OUT_D), jnp.float32),
    )(sh_table, yu_table)


def _make_sc_kernel(n_blocks):
    mesh = plsc.VectorSubcoreMesh(
        core_axis_name="c", subcore_axis_name="s",
        num_cores=NC, num_subcores=NS)

    @functools.partial(
        pl.kernel,
        out_type=jax.ShapeDtypeStruct((NW, n_blocks, BLK, OUT_D), jnp.float32),
        mesh=mesh,
        scratch_types=[
            pltpu.VMEM((n_blocks, BLK), jnp.int32),
            pltpu.VMEM((n_blocks, BLK), jnp.int32),
            pltpu.VMEM((BLK, OUT_D), jnp.float32),
            pltpu.VMEM((BLK, OUT_D), jnp.float32),
            pltpu.SemaphoreType.DMA,
            pltpu.SemaphoreType.DMA,
        ],
        compiler_params=pltpu.CompilerParams(use_tc_tiling_on_sc=False),
    )
    def sc_kernel(sidx_hbm, yidx_hbm, table_hbm, out_hbm,
                  comb_v, y_v, buf0, buf1, sem0, sem1):
        wid = lax.axis_index("s") * NC + lax.axis_index("c")

        pltpu.sync_copy(sidx_hbm.at[wid], comb_v)
        pltpu.sync_copy(yidx_hbm.at[wid], y_v)

        def fuse(t, _):
            i = t // (BLK // L)
            j = (t % (BLK // L)) * L
            comb_v[i, pl.ds(j, L)] = (
                comb_v[i, pl.ds(j, L)] * YU_V + y_v[i, pl.ds(j, L)])
            return 0
        lax.fori_loop(0, n_blocks * (BLK // L), fuse, 0)

        pltpu.async_copy(table_hbm.at[comb_v.at[0]], buf0, sem0)

        def step(p, _):
            g0 = 2 * p
            g1 = g0 + 1
            pltpu.async_copy(table_hbm.at[comb_v.at[g1]], buf1, sem1)
            pltpu.make_async_copy(table_hbm.at[comb_v.at[g0]], buf0,
                                  sem0).wait()
            pltpu.sync_copy(buf0, out_hbm.at[wid, g0])

            @pl.when(g0 + 2 < n_blocks)
            def _prefetch():
                pltpu.async_copy(table_hbm.at[comb_v.at[g0 + 2]], buf0, sem0)

            pltpu.make_async_copy(table_hbm.at[comb_v.at[g1]], buf1,
                                  sem1).wait()
            pltpu.sync_copy(buf1, out_hbm.at[wid, g1])
            return 0

        lax.fori_loop(0, n_blocks // 2, step, 0)

    return sc_kernel


def kernel(shengmu_indices, yunmu_indices, shengmu_table, yunmu_table):
    batch, seq = shengmu_indices.shape
    n = batch * seq
    assert n % (NW * BLK * 2) == 0
    n_blocks = n // (NW * BLK)

    table = _build_table(shengmu_table, yunmu_table)
    s = shengmu_indices.reshape(NW, n_blocks, BLK)
    y = yunmu_indices.reshape(NW, n_blocks, BLK)
    out = _make_sc_kernel(n_blocks)(s, y, table)
    return out.reshape(batch, seq, OUT_D)
